# baseline (speedup 1.0000x reference)
import functools

import jax
import jax.numpy as jnp
from jax import lax
from jax.experimental import pallas as pl
from jax.experimental.pallas import tpu as pltpu
from jax.experimental.pallas import tpu_sc as plsc

_NUM_STEPS = 1000
_DIM = 128
_BATCH = 16384

_NC = 2
_NS = 16
_NW = _NC * _NS
_CHUNKS = 4


def _mlp_table_kernel(e_ref, w1_ref, b1_ref, w2_ref, b2_ref, o_ref):
    dn = (((1,), (1,)), ((), ()))
    h = lax.dot_general(e_ref[...], w1_ref[...], dn,
                        preferred_element_type=jnp.float32)
    h = h + b1_ref[...]
    h = h * jax.nn.sigmoid(h)
    g = lax.dot_general(h, w2_ref[...], dn,
                        preferred_element_type=jnp.float32)
    g = g + b2_ref[...]
    o_ref[...] = g * jax.nn.sigmoid(g)


def _transform_table(embedding, W1, b1, W2, b2):
    return pl.pallas_call(
        _mlp_table_kernel,
        out_shape=jax.ShapeDtypeStruct((_NUM_STEPS, _DIM), jnp.float32),
    )(embedding, W1, b1.reshape(1, _DIM), W2, b2.reshape(1, _DIM))


def _make_sc_gather(batch, dim):
    b_per_w = batch // _NW
    rows_per_chunk = b_per_w // _CHUNKS
    mesh = plsc.VectorSubcoreMesh(core_axis_name="c", subcore_axis_name="s")

    @functools.partial(
        pl.kernel,
        mesh=mesh,
        out_type=jax.ShapeDtypeStruct((batch, dim), jnp.float32),
        scratch_types=[
            pltpu.VMEM((b_per_w,), jnp.int32),
            pltpu.VMEM((b_per_w, dim), jnp.float32),
            pltpu.VMEM_SHARED((_NUM_STEPS, dim), jnp.float32),
            pltpu.SemaphoreType.DMA,
        ],
    )
    def gather(table_hbm, idx_hbm, out_hbm, idx_v, rows_v, shared_t, sem):
        cid = lax.axis_index("c")
        sid = lax.axis_index("s")
        wid = sid * _NC + cid
        base = wid * b_per_w
        pltpu.sync_copy(idx_hbm.at[pl.ds(base, b_per_w)], idx_v)
        @pl.when(sid == 0)
        def _():
            pltpu.sync_copy(table_hbm, shared_t)
        plsc.subcore_barrier()
        pltpu.async_copy(shared_t.at[idx_v], rows_v, sem).wait()
        pltpu.sync_copy(rows_v, out_hbm.at[pl.ds(base, b_per_w)])

    return gather


_sc_gather = _make_sc_gather(_BATCH, _DIM)
_ROWS_PER_CHUNK = (_BATCH // _NW) // _CHUNKS


def kernel(diffusion_step, embedding, W1, b1, W2, b2):
    table = _transform_table(embedding, W1, b1, W2, b2)
    return _sc_gather(table, diffusion_step.astype(jnp.int32))


# speedup vs baseline: 1.0524x; 1.0524x over previous
import functools

import jax
import jax.numpy as jnp
from jax import lax
from jax.experimental import pallas as pl
from jax.experimental.pallas import tpu as pltpu
from jax.experimental.pallas import tpu_sc as plsc

_NUM_STEPS = 1000
_DIM = 128
_BATCH = 16384

_NC = 2
_NS = 16
_NW = _NC * _NS
_CHUNKS = 4


def _mlp_table_kernel(e_ref, w1_ref, b1_ref, w2_ref, b2_ref, o_ref):
    dn = (((1,), (1,)), ((), ()))
    h = lax.dot_general(e_ref[...], w1_ref[...], dn,
                        preferred_element_type=jnp.float32)
    h = h + b1_ref[...]
    h = h * jax.nn.sigmoid(h)
    g = lax.dot_general(h, w2_ref[...], dn,
                        preferred_element_type=jnp.float32)
    g = g + b2_ref[...]
    o_ref[...] = g * jax.nn.sigmoid(g)


def _transform_table(embedding, W1, b1, W2, b2):
    return pl.pallas_call(
        _mlp_table_kernel,
        out_shape=jax.ShapeDtypeStruct((_NUM_STEPS, _DIM), jnp.float32),
    )(embedding, W1, b1.reshape(1, _DIM), W2, b2.reshape(1, _DIM))


def _make_sc_gather(batch, dim):
    b_per_w = batch // _NW
    rows_per_chunk = b_per_w // _CHUNKS
    mesh = plsc.VectorSubcoreMesh(core_axis_name="c", subcore_axis_name="s")

    @functools.partial(
        pl.kernel,
        mesh=mesh,
        out_type=jax.ShapeDtypeStruct((batch, dim), jnp.float32),
        scratch_types=[
            pltpu.VMEM((_CHUNKS, rows_per_chunk), jnp.int32),
            [pltpu.VMEM((rows_per_chunk, dim), jnp.float32)
             for _ in range(_CHUNKS)],
            [pltpu.SemaphoreType.DMA for _ in range(_CHUNKS)],
            pltpu.VMEM_SHARED((_NUM_STEPS, dim), jnp.float32),
            pltpu.SemaphoreType.DMA,
            pltpu.SemaphoreType.DMA,
        ],
    )
    def gather(table_hbm, idx_hbm, out_hbm, idx_v, bufs, gsems, shared_t,
               wsem, isem):
        cid = lax.axis_index("c")
        sid = lax.axis_index("s")
        wid = sid * _NC + cid
        base = wid * b_per_w
        idx_cp = pltpu.async_copy(
            idx_hbm.at[pl.ds(wid * _CHUNKS, _CHUNKS)], idx_v, isem)
        @pl.when(sid == 0)
        def _():
            pltpu.sync_copy(table_hbm, shared_t)
        idx_cp.wait()
        plsc.subcore_barrier()
        gathers = [
            pltpu.async_copy(shared_t.at[idx_v.at[c]], bufs[c], gsems[c])
            for c in range(_CHUNKS)
        ]
        writes = [None] * _CHUNKS
        for c in range(_CHUNKS):
            gathers[c].wait()
            writes[c] = pltpu.async_copy(
                bufs[c],
                out_hbm.at[pl.ds(base + c * rows_per_chunk, rows_per_chunk)],
                wsem)
        for c in range(_CHUNKS):
            writes[c].wait()

    return gather


_sc_gather = _make_sc_gather(_BATCH, _DIM)
_ROWS_PER_CHUNK = (_BATCH // _NW) // _CHUNKS


def kernel(diffusion_step, embedding, W1, b1, W2, b2):
    table = _transform_table(embedding, W1, b1, W2, b2)
    idx = diffusion_step.astype(jnp.int32).reshape(
        _BATCH // _ROWS_PER_CHUNK, _ROWS_PER_CHUNK)
    return _sc_gather(table, idx)
